# Initial kernel scaffold; baseline (speedup 1.0000x reference)
import jax
import jax.numpy as jnp
from jax.experimental import pallas as pl


def kernel(x, W1, g1, b1, W2, g2, b2, W3, g3, b3):
    raise NotImplementedError("write your pallas kernel here")


# speedup vs baseline: 30.9835x; 30.9835x over previous
import functools

import jax
import jax.numpy as jnp
from jax.experimental import pallas as pl

_K = 20


def _attn_conv_kernel(xt_ref, w_ref, y_ref, *, k, tq):
    q = pl.program_id(1)
    xt = xt_ref[0]
    tile = xt_ref[0, pl.ds(q * tq, tq), :]
    coln = jnp.sum(xt * xt, axis=1)
    rown = jnp.sum(tile * tile, axis=1, keepdims=True)
    d = 2.0 * jax.lax.dot_general(
        tile, xt, (((1,), (1,)), ((), ())),
        preferred_element_type=jnp.float32)
    d = d - rown - coln[None, :]

    dw = d
    s0 = None
    t = None
    for it in range(k):
        m = jnp.max(dw, axis=1, keepdims=True)
        if it == 0:
            s0 = m
        if it == k - 1:
            t = m
        else:
            dw = jnp.where(dw == m, -jnp.inf, dw)

    w = jnp.where(d >= t, jnp.exp(d - s0), 0.0)
    den = jnp.sum(w, axis=1, keepdims=True)
    agg = jax.lax.dot_general(
        w, xt, (((1,), (0,)), ((), ())),
        preferred_element_type=jnp.float32)
    wnn = agg / den - tile
    att = jnp.concatenate([tile, wnn], axis=1)
    y = jax.lax.dot_general(
        att, w_ref[...], (((1,), (1,)), ((), ())),
        preferred_element_type=jnp.float32)
    y_ref[0] = y


def _bn_lrelu_kernel(y_ref, g_ref, b_ref, x_ref):
    y = y_ref[...]
    c = y.shape[-1]
    y2 = y.reshape(-1, c)
    m = jnp.mean(y2, axis=0, keepdims=True)
    v = jnp.mean((y2 - m) ** 2, axis=0, keepdims=True)
    xn = (y2 - m) / jnp.sqrt(v + 1e-5)
    xn = xn * g_ref[...] + b_ref[...]
    x = jnp.where(xn > 0, xn, 0.2 * xn)
    x_ref[...] = x.reshape(y.shape)


def _round(xt, w, g, b, tq):
    bsz, n, c = xt.shape
    cout = w.shape[0]
    y = pl.pallas_call(
        functools.partial(_attn_conv_kernel, k=_K, tq=tq),
        grid=(bsz, n // tq),
        in_specs=[
            pl.BlockSpec((1, n, c), lambda bi, qi: (bi, 0, 0)),
            pl.BlockSpec((cout, 2 * c), lambda bi, qi: (0, 0)),
        ],
        out_specs=pl.BlockSpec((1, tq, cout), lambda bi, qi: (bi, qi, 0)),
        out_shape=jax.ShapeDtypeStruct((bsz, n, cout), jnp.float32),
    )(xt, w)
    x = pl.pallas_call(
        _bn_lrelu_kernel,
        in_specs=[
            pl.BlockSpec((bsz, n, cout), lambda: (0, 0, 0)),
            pl.BlockSpec((1, cout), lambda: (0, 0)),
            pl.BlockSpec((1, cout), lambda: (0, 0)),
        ],
        out_specs=pl.BlockSpec((bsz, n, cout), lambda: (0, 0, 0)),
        out_shape=jax.ShapeDtypeStruct((bsz, n, cout), jnp.float32),
    )(y, g.reshape(1, cout), b.reshape(1, cout))
    return x


def kernel(x, W1, g1, b1, W2, g2, b2, W3, g3, b3):
    xt = jnp.transpose(x, (0, 2, 1))
    x1 = _round(xt, W1, g1, b1, 256)
    x2 = _round(x1, W2, g2, b2, 256)
    x3 = _round(x2, W3, g3, b3, 256)
    return (jnp.transpose(x1, (0, 2, 1)),
            jnp.transpose(x2, (0, 2, 1)),
            jnp.transpose(x3, (0, 2, 1)))


# speedup vs baseline: 38.3436x; 1.2375x over previous
import functools

import jax
import jax.numpy as jnp
from jax.experimental import pallas as pl

_K = 20


def _bitonic_pairs(n):
    pairs = []
    k = 2
    while k <= n:
        j = k >> 1
        while j >= 1:
            for i in range(n):
                l = i ^ j
                if l > i:
                    pairs.append((i, l, (i & k) == 0))
            j >>= 1
        k <<= 1
    return pairs


def _attn_conv_kernel(xt_ref, w_ref, y_ref, *, k, tq):
    q = pl.program_id(1)
    xt = xt_ref[0]
    tile = xt_ref[0, pl.ds(q * tq, tq), :]
    coln = jnp.sum(xt * xt, axis=1)
    rown = jnp.sum(tile * tile, axis=1, keepdims=True)
    d = 2.0 * jax.lax.dot_general(
        tile, xt, (((1,), (1,)), ((), ())),
        preferred_element_type=jnp.float32)
    d = d - rown - coln[None, :]

    n = d.shape[1]
    nchunk = n // 128
    ch = [d[:, c * 128:(c + 1) * 128] for c in range(nchunk)]
    for i, l, up in _bitonic_pairs(nchunk):
        hi = jnp.maximum(ch[i], ch[l])
        lo = jnp.minimum(ch[i], ch[l])
        ch[i], ch[l] = (hi, lo) if up else (lo, hi)
    neg_inf = jnp.full_like(ch[0], -jnp.inf)
    s0 = None
    t = None
    for it in range(k):
        m = jnp.max(ch[0], axis=1, keepdims=True)
        if it == 0:
            s0 = m
        if it == k - 1:
            t = m
        else:
            mask = ch[0] == m
            depth = min(nchunk - 1, k - 2 - it)
            for j in range(depth + 1):
                nxt = ch[j + 1] if j + 1 < nchunk else neg_inf
                ch[j] = jnp.where(mask, nxt, ch[j])

    w = jnp.where(d >= t, jnp.exp(d - s0), 0.0)
    den = jnp.sum(w, axis=1, keepdims=True)
    agg = jax.lax.dot_general(
        w, xt, (((1,), (0,)), ((), ())),
        preferred_element_type=jnp.float32)
    wnn = agg / den - tile
    att = jnp.concatenate([tile, wnn], axis=1)
    y = jax.lax.dot_general(
        att, w_ref[...], (((1,), (1,)), ((), ())),
        preferred_element_type=jnp.float32)
    y_ref[0] = y


def _bn_lrelu_kernel(y_ref, g_ref, b_ref, x_ref):
    y = y_ref[...]
    c = y.shape[-1]
    y2 = y.reshape(-1, c)
    m = jnp.mean(y2, axis=0, keepdims=True)
    v = jnp.mean((y2 - m) ** 2, axis=0, keepdims=True)
    xn = (y2 - m) / jnp.sqrt(v + 1e-5)
    xn = xn * g_ref[...] + b_ref[...]
    x = jnp.where(xn > 0, xn, 0.2 * xn)
    x_ref[...] = x.reshape(y.shape)


def _round(xt, w, g, b, tq):
    bsz, n, c = xt.shape
    cout = w.shape[0]
    y = pl.pallas_call(
        functools.partial(_attn_conv_kernel, k=_K, tq=tq),
        grid=(bsz, n // tq),
        in_specs=[
            pl.BlockSpec((1, n, c), lambda bi, qi: (bi, 0, 0)),
            pl.BlockSpec((cout, 2 * c), lambda bi, qi: (0, 0)),
        ],
        out_specs=pl.BlockSpec((1, tq, cout), lambda bi, qi: (bi, qi, 0)),
        out_shape=jax.ShapeDtypeStruct((bsz, n, cout), jnp.float32),
    )(xt, w)
    x = pl.pallas_call(
        _bn_lrelu_kernel,
        in_specs=[
            pl.BlockSpec((bsz, n, cout), lambda: (0, 0, 0)),
            pl.BlockSpec((1, cout), lambda: (0, 0)),
            pl.BlockSpec((1, cout), lambda: (0, 0)),
        ],
        out_specs=pl.BlockSpec((bsz, n, cout), lambda: (0, 0, 0)),
        out_shape=jax.ShapeDtypeStruct((bsz, n, cout), jnp.float32),
    )(y, g.reshape(1, cout), b.reshape(1, cout))
    return x


def kernel(x, W1, g1, b1, W2, g2, b2, W3, g3, b3):
    xt = jnp.transpose(x, (0, 2, 1))
    x1 = _round(xt, W1, g1, b1, 256)
    x2 = _round(x1, W2, g2, b2, 256)
    x3 = _round(x2, W3, g3, b3, 256)
    return (jnp.transpose(x1, (0, 2, 1)),
            jnp.transpose(x2, (0, 2, 1)),
            jnp.transpose(x3, (0, 2, 1)))
